# baseline (speedup 1.0000x reference)
import jax
import jax.numpy as jnp
from jax import lax
from jax.experimental import pallas as pl
from jax.experimental.pallas import tpu as pltpu
from jax.experimental.pallas import tpu_sc as plsc

N_ROWS = 1_000_000
D = 64
B = 16384
NC = 2
NS = 16
NW = NC * NS
TPW = B // NW
PACK = 2 * D
NP = N_ROWS // 2
OP = B // 2
CHUNK = 64
NCHUNK = TPW // CHUNK
PPC = CHUNK // 2
NBUF = 2


def _body(w2, idx_hbm, out2, idx_v, tidx_v, stg0, stg1, ob0, ob1,
          gs0, gs1, ws0, ws1):
    stg = [stg0, stg1]
    ob = [ob0, ob1]
    gs = [gs0, gs1]
    ws = [ws0, ws1]

    wid = lax.axis_index("s") * NC + lax.axis_index("c")
    base = wid * TPW
    pltpu.sync_copy(idx_hbm.at[pl.ds(base, TPW)], idx_v)
    for g in range(TPW // 16):
        tidx_v[pl.ds(g * 16, 16)] = idx_v[pl.ds(g * 16, 16)] >> 1

    def fire_gather(chunk, b):
        pltpu.async_copy(
            w2.at[tidx_v.at[pl.ds(chunk * CHUNK, CHUNK)]], stg[b], gs[b])

    for b in range(NBUF):
        fire_gather(b, b)

    @pl.loop(0, NCHUNK, step=NBUF)
    def _chunks(i):
        for b in range(NBUF):
            chunk = i + b
            pltpu.make_async_copy(
                w2.at[tidx_v.at[pl.ds(chunk * CHUNK, CHUNK)]],
                stg[b], gs[b]).wait()
            @pl.when(chunk >= NBUF)
            def _():
                pltpu.make_async_copy(
                    ob[b], out2.at[pl.ds(0, PPC)], ws[b]).wait()
            cbase = chunk * CHUNK
            for g in range(CHUNK // 16):
                jv = lax.iota(jnp.int32, 16) + g * 16
                iv = idx_v[pl.ds(cbase + g * 16, 16)]
                hv = (iv & 1) * D
                pv = jv >> 1
                qv = (jv & 1) * D
                for c in range(D):
                    cv = jnp.full((16,), c, jnp.int32)
                    vals = plsc.load_gather(stg[b], [jv, hv + cv])
                    plsc.store_scatter(ob[b], [pv, qv + cv], vals)
            pltpu.async_copy(
                ob[b], out2.at[pl.ds(wid * (TPW // 2) + chunk * PPC, PPC)],
                ws[b])
            @pl.when(chunk + NBUF < NCHUNK)
            def _():
                fire_gather(chunk + NBUF, b)

    for b in range(NBUF):
        pltpu.make_async_copy(ob[b], out2.at[pl.ds(0, PPC)], ws[b]).wait()


def kernel(input, weight):
    idx = input.astype(jnp.int32)
    w2 = weight.reshape(NP, PACK)
    mesh = plsc.VectorSubcoreMesh(core_axis_name="c", subcore_axis_name="s")
    k = pl.kernel(
        _body,
        out_type=jax.ShapeDtypeStruct((OP, PACK), jnp.float32),
        mesh=mesh,
        scratch_types=[
            pltpu.VMEM((TPW,), jnp.int32),
            pltpu.VMEM((TPW,), jnp.int32),
            pltpu.VMEM((CHUNK, PACK), jnp.float32),
            pltpu.VMEM((CHUNK, PACK), jnp.float32),
            pltpu.VMEM((PPC, PACK), jnp.float32),
            pltpu.VMEM((PPC, PACK), jnp.float32),
            pltpu.SemaphoreType.DMA,
            pltpu.SemaphoreType.DMA,
            pltpu.SemaphoreType.DMA,
            pltpu.SemaphoreType.DMA,
        ],
        compiler_params=pltpu.CompilerParams(needs_layout_passes=False),
    )
    out2 = k(w2, idx)
    return out2.reshape(B, D)
